# baseline (speedup 1.0000x reference)
import functools

import jax
import jax.numpy as jnp
from jax import lax
from jax.experimental import pallas as pl
from jax.experimental.pallas import tpu as pltpu
from jax.experimental.pallas import tpu_sc as plsc

_BATCH = 16384
_ROWS = 1000000
_NW = 32
_NPW = _BATCH // _NW
_CSZ = 128
_NBLK = _NPW // _CSZ

_TPH = 7813
_NTILES = 2 * _TPH
_TPW = 488
_TREM = _TPH - 16 * _TPW
_WIN = 7812 * 1024 + 128
_HSTRIDE = _TPH * 1024

_CK = 32
_CHUNKS = [_CK] * (_TPW // _CK) + ([_TPW % _CK] if _TPW % _CK else [])
_NBUF = 3


def _full(v):
    return jnp.full((16,), v, jnp.int32)


def _retile(tbl_hbm, xt_hbm, w_hbm, b_hbm,
            t2_hbm, idx_hbm, acc_hbm, wp_hbm,
            users_v, lb_v, idx_v, acc_v, wp_v, wf_v, bf_v, vms, sins, souts):
    wid = lax.axis_index("s") * 2 + lax.axis_index("c")
    fb = wid >> 4
    lw = wid & 15
    base = lw * _TPW

    def fire_ins(ch):
        vm, sin = vms[ch % _NBUF], sins[ch % _NBUF]
        t0 = base + ch * _CK
        for k in range(_CHUNKS[ch]):
            pltpu.async_copy(
                tbl_hbm.at[pl.ds(fb * 8, 8),
                           pl.ds(pl.multiple_of((t0 + k) * 128, 128), 128)],
                vm.at[k], sin)

    def wait_ins(ch):
        pltpu.make_async_copy(
            t2_hbm.at[pl.ds(0, _CHUNKS[ch])],
            vms[ch % _NBUF].at[pl.ds(0, _CHUNKS[ch])], sins[ch % _NBUF]).wait()

    def fire_out(ch):
        g0 = fb * _TPH + base + ch * _CK
        pltpu.async_copy(
            vms[ch % _NBUF].at[pl.ds(0, _CHUNKS[ch])],
            t2_hbm.at[pl.ds(g0, _CHUNKS[ch])], souts[ch % _NBUF])

    def wait_out(ch):
        pltpu.make_async_copy(
            t2_hbm.at[pl.ds(0, _CHUNKS[ch])],
            vms[ch % _NBUF].at[pl.ds(0, _CHUNKS[ch])], souts[ch % _NBUF]).wait()

    n = len(_CHUNKS)
    for ch in range(min(_NBUF - 1, n)):
        fire_ins(ch)

    bbase = wid * _NPW
    pltpu.sync_copy(xt_hbm.at[pl.ds(bbase, _NPW)],
                    users_v.at[pl.ds(0, _NPW)])
    pltpu.sync_copy(xt_hbm.at[pl.ds(_BATCH + bbase, _NPW)],
                    users_v.at[pl.ds(_NPW, _NPW)])
    pltpu.sync_copy(xt_hbm.at[pl.ds(4 * _BATCH + bbase, _NPW)], lb_v)
    pltpu.sync_copy(w_hbm, wf_v)
    pltpu.sync_copy(b_hbm, bf_v)

    wlbl = (plsc.load_gather(wf_v, [_full(65)])
            - plsc.load_gather(wf_v, [_full(32)]))
    wdb = (plsc.load_gather(bf_v, [_full(1)])
           - plsc.load_gather(bf_v, [_full(0)]))

    def build(g, carry):
        for t in range(8):
            u = users_v[pl.ds(g * _CSZ + t * 16, 16)].astype(jnp.int32)
            v = ((u >> 7) << 10) | (u & 127)
            idx_v[pl.ds(g * _CSZ + t * 16, 16)] = v
        return carry

    lax.fori_loop(0, 2 * _NBLK, build, 0)

    def seed(g, carry):
        for t in range(8):
            e = g * _CSZ + t * 16
            acc_v[pl.ds(e, 16)] = lb_v[pl.ds(e, 16)] * wlbl + wdb
        return carry

    lax.fori_loop(0, _NBLK, seed, 0)

    pltpu.sync_copy(idx_v, idx_hbm.at[pl.ds(wid * 2 * _NPW, 2 * _NPW)])
    pltpu.sync_copy(acc_v, acc_hbm.at[pl.ds(bbase, _NPW)])

    @pl.when(wid == 0)
    def _wp():
        for c in range(32):
            wp_v[pl.ds(c * 16, 16)] = (
                plsc.load_gather(wf_v, [_full(33 + c)])
                - plsc.load_gather(wf_v, [_full(c)]))
        pltpu.sync_copy(wp_v, wp_hbm)

    for ch in range(n):
        wait_ins(ch)
        fire_out(ch)
        nx = ch + _NBUF - 1
        if nx < n:
            if nx - _NBUF >= 0:
                wait_out(nx - _NBUF)
            fire_ins(nx)
    for ch in range(n - min(_NBUF, n), n):
        wait_out(ch)

    @pl.when(lw < _TREM)
    def _rem():
        rb = 16 * _TPW + lw
        pltpu.async_copy(
            tbl_hbm.at[pl.ds(fb * 8, 8),
                       pl.ds(pl.multiple_of(rb * 128, 128), 128)],
            vms[0].at[0], sins[0])
        pltpu.make_async_copy(t2_hbm.at[pl.ds(0, 1)],
                              vms[0].at[pl.ds(0, 1)], sins[0]).wait()
        pltpu.async_copy(vms[0].at[pl.ds(0, 1)],
                         t2_hbm.at[pl.ds(fb * _TPH + rb, 1)], souts[0])
        pltpu.make_async_copy(t2_hbm.at[pl.ds(0, 1)],
                              vms[0].at[pl.ds(0, 1)], souts[0]).wait()


def _body(tbl_hbm, idx_hbm, acc_hbm, wp_hbm, out_hbm,
          idx_v, acc_v, wp_v, val_v, o0_v, o1_v, sem):
    wid = lax.axis_index("s") * 2 + lax.axis_index("c")
    base = wid * _NPW

    pltpu.sync_copy(idx_hbm.at[pl.ds(wid * 2 * _NPW, 2 * _NPW)], idx_v)
    pltpu.sync_copy(acc_hbm.at[pl.ds(base, _NPW)], acc_v)
    pltpu.sync_copy(wp_hbm, wp_v)

    for sel in range(2):
        for blk in range(_NBLK):
            isl = idx_v.at[pl.ds((sel * _NBLK + blk) * _CSZ, _CSZ)]
            for c in range(16):
                cbase = (c >> 3) * _HSTRIDE + (c & 7) * 128
                pltpu.async_copy(
                    tbl_hbm.at[pl.ds(cbase, _WIN)].at[isl],
                    val_v.at[pl.ds(((sel * 16 + c) * _NBLK + blk) * _CSZ,
                                   _CSZ)],
                    sem)
    pltpu.make_async_copy(tbl_hbm.at[pl.ds(0, 32 * _NPW)], val_v, sem).wait()

    wv = [wp_v[pl.ds(c * 16, 16)] for c in range(32)]

    def compute_blk(blk, carry):
        for t in range(8):
            e = blk * 128 + t * 16
            acc = acc_v[pl.ds(e, 16)]
            for c in range(16):
                vl = val_v[pl.ds((c * _NBLK + blk) * _CSZ + t * 16, 16)]
                acc = acc + vl * wv[c]
            for c in range(16):
                vc = val_v[pl.ds(((16 + c) * _NBLK + blk) * _CSZ + t * 16, 16)]
                acc = acc + vc * wv[16 + c]
            e0 = jnp.exp(jnp.minimum(-acc, 0.0))
            e1 = jnp.exp(jnp.minimum(acc, 0.0))
            rz = 1.0 / (e0 + e1)
            o0_v[pl.ds(e, 16)] = e0 * rz
            o1_v[pl.ds(e, 16)] = e1 * rz
        return carry

    lax.fori_loop(0, _NBLK, compute_blk, 0)

    pltpu.sync_copy(o0_v, out_hbm.at[pl.ds(base, _NPW)])
    pltpu.sync_copy(o1_v, out_hbm.at[pl.ds(_BATCH + base, _NPW)])


@functools.partial(jax.jit, static_argnums=())
def _run(xt_flat, tbl_t, w_flat, b):
    mesh = plsc.VectorSubcoreMesh(core_axis_name="c", subcore_axis_name="s")
    retile = pl.kernel(
        _retile,
        out_type=[
            jax.ShapeDtypeStruct((_NTILES, 8, 128), jnp.float32),
            jax.ShapeDtypeStruct((2 * _BATCH,), jnp.int32),
            jax.ShapeDtypeStruct((_BATCH,), jnp.float32),
            jax.ShapeDtypeStruct((34 * 16,), jnp.float32),
        ],
        mesh=mesh,
        scratch_types=[
            pltpu.VMEM((2 * _NPW,), jnp.float32),
            pltpu.VMEM((_NPW,), jnp.float32),
            pltpu.VMEM((2 * _NPW,), jnp.int32),
            pltpu.VMEM((_NPW,), jnp.float32),
            pltpu.VMEM((34 * 16,), jnp.float32),
            pltpu.VMEM((66,), jnp.float32),
            pltpu.VMEM((2,), jnp.float32),
            [pltpu.VMEM((_CK, 8, 128), jnp.float32) for _ in range(_NBUF)],
            [pltpu.SemaphoreType.DMA for _ in range(_NBUF)],
            [pltpu.SemaphoreType.DMA for _ in range(_NBUF)],
        ],
        compiler_params=pltpu.CompilerParams(
            needs_layout_passes=False, use_tc_tiling_on_sc=True),
    )
    t2, idx, acc, wp = retile(tbl_t, xt_flat, w_flat, b)
    f = pl.kernel(
        _body,
        out_type=jax.ShapeDtypeStruct((2 * _BATCH,), jnp.float32),
        mesh=mesh,
        scratch_types=[
            pltpu.VMEM((2 * _NPW,), jnp.int32),
            pltpu.VMEM((_NPW,), jnp.float32),
            pltpu.VMEM((34 * 16,), jnp.float32),
            pltpu.VMEM((32 * _NPW,), jnp.float32),
            pltpu.VMEM((_NPW,), jnp.float32),
            pltpu.VMEM((_NPW,), jnp.float32),
            pltpu.SemaphoreType.DMA,
        ],
        compiler_params=pltpu.CompilerParams(
            needs_layout_passes=False, use_tc_tiling_on_sc=False),
    )
    return f(t2.reshape(-1), idx, acc, wp)


def kernel(x, table, W, b):
    xt = x.transpose(2, 1, 0).reshape(-1)
    out = _run(xt, table.T, W.reshape(-1), b)
    return out.reshape(2, _BATCH).transpose(1, 0)


# speedup vs baseline: 1.0373x; 1.0373x over previous
import functools

import jax
import jax.numpy as jnp
from jax import lax
from jax.experimental import pallas as pl
from jax.experimental.pallas import tpu as pltpu
from jax.experimental.pallas import tpu_sc as plsc

_BATCH = 16384
_ROWS = 1000000
_NW = 32
_NPW = _BATCH // _NW
_CSZ = 128
_NBLK = _NPW // _CSZ

_TPH = 7813
_NTILES = 2 * _TPH
_TPW = 488
_TREM = _TPH - 16 * _TPW
_WIN = 7812 * 1024 + 128
_HSTRIDE = _TPH * 1024


_CK = 24
_CHUNKS = [_CK] * (_TPW // _CK) + ([_TPW % _CK] if _TPW % _CK else [])
_NBUF = 4


def _retile(tbl_hbm, t2_hbm, vms, sins, souts):
    wid = lax.axis_index("s") * 2 + lax.axis_index("c")
    fb = wid >> 4
    lw = wid & 15
    base = lw * _TPW

    def fire_ins(ch):
        vm, sin = vms[ch % _NBUF], sins[ch % _NBUF]
        t0 = base + ch * _CK
        for k in range(_CHUNKS[ch]):
            pltpu.async_copy(
                tbl_hbm.at[pl.ds(fb * 8, 8),
                           pl.ds(pl.multiple_of((t0 + k) * 128, 128), 128)],
                vm.at[k], sin)

    def wait_ins(ch):
        pltpu.make_async_copy(
            t2_hbm.at[pl.ds(0, _CHUNKS[ch])],
            vms[ch % _NBUF].at[pl.ds(0, _CHUNKS[ch])], sins[ch % _NBUF]).wait()

    def fire_out(ch):
        g0 = fb * _TPH + base + ch * _CK
        pltpu.async_copy(
            vms[ch % _NBUF].at[pl.ds(0, _CHUNKS[ch])],
            t2_hbm.at[pl.ds(g0, _CHUNKS[ch])], souts[ch % _NBUF])

    def wait_out(ch):
        pltpu.make_async_copy(
            t2_hbm.at[pl.ds(0, _CHUNKS[ch])],
            vms[ch % _NBUF].at[pl.ds(0, _CHUNKS[ch])], souts[ch % _NBUF]).wait()

    n = len(_CHUNKS)
    out_pending = []
    for ch in range(min(_NBUF - 1, n)):
        fire_ins(ch)
    for ch in range(n):
        wait_ins(ch)
        fire_out(ch)
        out_pending.append(ch)
        nx = ch + _NBUF - 1
        if nx < n:
            if nx - _NBUF >= 0:
                wait_out(nx - _NBUF)
                out_pending.remove(nx - _NBUF)
            fire_ins(nx)
    for ch in out_pending:
        wait_out(ch)

    @pl.when(lw < _TREM)
    def _rem():
        rb = 16 * _TPW + lw
        pltpu.async_copy(
            tbl_hbm.at[pl.ds(fb * 8, 8),
                       pl.ds(pl.multiple_of(rb * 128, 128), 128)],
            vms[0].at[0], sins[0])
        pltpu.make_async_copy(t2_hbm.at[pl.ds(0, 1)],
                              vms[0].at[pl.ds(0, 1)], sins[0]).wait()
        pltpu.async_copy(vms[0].at[pl.ds(0, 1)],
                         t2_hbm.at[pl.ds(fb * _TPH + rb, 1)], souts[0])
        pltpu.make_async_copy(t2_hbm.at[pl.ds(0, 1)],
                              vms[0].at[pl.ds(0, 1)], souts[0]).wait()


def _body(xt_hbm, tbl_hbm, wp_hbm, out_hbm,
          users_v, lb_v, wp_v, idx_v, val_v, o0_v, o1_v, sems):
    wid = lax.axis_index("s") * 2 + lax.axis_index("c")
    base = wid * _NPW

    pltpu.sync_copy(xt_hbm.at[pl.ds(base, _NPW)], users_v.at[pl.ds(0, _NPW)])
    pltpu.sync_copy(xt_hbm.at[pl.ds(_BATCH + base, _NPW)],
                    users_v.at[pl.ds(_NPW, _NPW)])
    pltpu.sync_copy(xt_hbm.at[pl.ds(4 * _BATCH + base, _NPW)], lb_v)
    pltpu.sync_copy(wp_hbm, wp_v)

    def build(g, carry):
        for t in range(8):
            u = users_v[pl.ds(g * _CSZ + t * 16, 16)].astype(jnp.int32)
            v = ((u >> 7) << 10) | (u & 127)
            idx_v[pl.ds(g * _CSZ + t * 16, 16)] = v
        return carry

    lax.fori_loop(0, 2 * _NBLK, build, 0)

    for blk in range(_NBLK):
        for sel in range(2):
            isl = idx_v.at[pl.ds((sel * _NBLK + blk) * _CSZ, _CSZ)]
            for c in range(16):
                cbase = (c >> 3) * _HSTRIDE + (c & 7) * 128
                pltpu.async_copy(
                    tbl_hbm.at[pl.ds(cbase, _WIN)].at[isl],
                    val_v.at[pl.ds(((sel * 16 + c) * _NBLK + blk) * _CSZ,
                                   _CSZ)],
                    sems[blk])

    wlbl = wp_v[pl.ds(32 * 16, 16)]
    wdb = wp_v[pl.ds(33 * 16, 16)]
    wv = [wp_v[pl.ds(c * 16, 16)] for c in range(32)]

    for blk in range(_NBLK):
        pltpu.make_async_copy(xt_hbm.at[pl.ds(0, 32 * _CSZ)],
                              val_v.at[pl.ds(0, 32 * _CSZ)], sems[blk]).wait()
        for t in range(8):
            e = blk * 128 + t * 16
            acc = lb_v[pl.ds(e, 16)] * wlbl + wdb
            for c in range(16):
                vl = val_v[pl.ds((c * _NBLK + blk) * _CSZ + t * 16, 16)]
                acc = acc + vl * wv[c]
            for c in range(16):
                vc = val_v[pl.ds(((16 + c) * _NBLK + blk) * _CSZ + t * 16, 16)]
                acc = acc + vc * wv[16 + c]
            e0 = jnp.exp(jnp.minimum(-acc, 0.0))
            e1 = jnp.exp(jnp.minimum(acc, 0.0))
            rz = 1.0 / (e0 + e1)
            o0_v[pl.ds(e, 16)] = e0 * rz
            o1_v[pl.ds(e, 16)] = e1 * rz

    pltpu.sync_copy(o0_v, out_hbm.at[pl.ds(base, _NPW)])
    pltpu.sync_copy(o1_v, out_hbm.at[pl.ds(_BATCH + base, _NPW)])


@functools.partial(jax.jit, static_argnums=())
def _run(xt_flat, tbl_t, wp):
    mesh = plsc.VectorSubcoreMesh(core_axis_name="c", subcore_axis_name="s")
    retile = pl.kernel(
        _retile,
        out_type=jax.ShapeDtypeStruct((_NTILES, 8, 128), jnp.float32),
        mesh=mesh,
        scratch_types=[
            [pltpu.VMEM((_CK, 8, 128), jnp.float32) for _ in range(_NBUF)],
            [pltpu.SemaphoreType.DMA for _ in range(_NBUF)],
            [pltpu.SemaphoreType.DMA for _ in range(_NBUF)],
        ],
        compiler_params=pltpu.CompilerParams(use_tc_tiling_on_sc=True),
    )
    t2 = retile(tbl_t)
    f = pl.kernel(
        _body,
        out_type=jax.ShapeDtypeStruct((2 * _BATCH,), jnp.float32),
        mesh=mesh,
        scratch_types=[
            pltpu.VMEM((2 * _NPW,), jnp.float32),
            pltpu.VMEM((_NPW,), jnp.float32),
            pltpu.VMEM((34 * 16,), jnp.float32),
            pltpu.VMEM((2 * _NPW,), jnp.int32),
            pltpu.VMEM((32 * _NPW,), jnp.float32),
            pltpu.VMEM((_NPW,), jnp.float32),
            pltpu.VMEM((_NPW,), jnp.float32),
            [pltpu.SemaphoreType.DMA for _ in range(_NBLK)],
        ],
        compiler_params=pltpu.CompilerParams(
            needs_layout_passes=False, use_tc_tiling_on_sc=False),
    )
    return f(xt_flat, t2.reshape(-1), wp)


def kernel(x, table, W, b):
    wd = W[1] - W[0]
    wp = jnp.concatenate([
        jnp.broadcast_to(wd[:32, None], (32, 16)),
        jnp.full((1, 16), wd[32], jnp.float32),
        jnp.full((1, 16), b[1] - b[0], jnp.float32),
    ], axis=0).reshape(-1)
    xt = x.transpose(2, 1, 0).reshape(-1)
    out = _run(xt, table.T, wp)
    return out.reshape(2, _BATCH).transpose(1, 0)


# speedup vs baseline: 1.0403x; 1.0029x over previous
import functools

import jax
import jax.numpy as jnp
from jax import lax
from jax.experimental import pallas as pl
from jax.experimental.pallas import tpu as pltpu
from jax.experimental.pallas import tpu_sc as plsc

_BATCH = 16384
_ROWS = 1000000
_NW = 32
_NPW = _BATCH // _NW
_CSZ = 128
_NBLK = _NPW // _CSZ

_TPH = 7813
_NTILES = 2 * _TPH
_TPW = 488
_TREM = _TPH - 16 * _TPW
_WIN = 7812 * 1024 + 128
_HSTRIDE = _TPH * 1024


_CK = 16
_CHUNKS = [_CK] * (_TPW // _CK) + ([_TPW % _CK] if _TPW % _CK else [])
_NBUF = 6


def _retile(tbl_hbm, t2_hbm, vms, sins, souts):
    wid = lax.axis_index("s") * 2 + lax.axis_index("c")
    fb = wid >> 4
    lw = wid & 15
    base = lw * _TPW

    def fire_ins(ch):
        vm, sin = vms[ch % _NBUF], sins[ch % _NBUF]
        t0 = base + ch * _CK
        for k in range(_CHUNKS[ch]):
            pltpu.async_copy(
                tbl_hbm.at[pl.ds(fb * 8, 8),
                           pl.ds(pl.multiple_of((t0 + k) * 128, 128), 128)],
                vm.at[k], sin)

    def wait_ins(ch):
        pltpu.make_async_copy(
            t2_hbm.at[pl.ds(0, _CHUNKS[ch])],
            vms[ch % _NBUF].at[pl.ds(0, _CHUNKS[ch])], sins[ch % _NBUF]).wait()

    def fire_out(ch):
        g0 = fb * _TPH + base + ch * _CK
        pltpu.async_copy(
            vms[ch % _NBUF].at[pl.ds(0, _CHUNKS[ch])],
            t2_hbm.at[pl.ds(g0, _CHUNKS[ch])], souts[ch % _NBUF])

    def wait_out(ch):
        pltpu.make_async_copy(
            t2_hbm.at[pl.ds(0, _CHUNKS[ch])],
            vms[ch % _NBUF].at[pl.ds(0, _CHUNKS[ch])], souts[ch % _NBUF]).wait()

    n = len(_CHUNKS)
    out_pending = []
    for ch in range(min(_NBUF - 1, n)):
        fire_ins(ch)
    for ch in range(n):
        wait_ins(ch)
        fire_out(ch)
        out_pending.append(ch)
        nx = ch + _NBUF - 1
        if nx < n:
            if nx - _NBUF >= 0:
                wait_out(nx - _NBUF)
                out_pending.remove(nx - _NBUF)
            fire_ins(nx)
    for ch in out_pending:
        wait_out(ch)

    @pl.when(lw < _TREM)
    def _rem():
        rb = 16 * _TPW + lw
        pltpu.async_copy(
            tbl_hbm.at[pl.ds(fb * 8, 8),
                       pl.ds(pl.multiple_of(rb * 128, 128), 128)],
            vms[0].at[0], sins[0])
        pltpu.make_async_copy(t2_hbm.at[pl.ds(0, 1)],
                              vms[0].at[pl.ds(0, 1)], sins[0]).wait()
        pltpu.async_copy(vms[0].at[pl.ds(0, 1)],
                         t2_hbm.at[pl.ds(fb * _TPH + rb, 1)], souts[0])
        pltpu.make_async_copy(t2_hbm.at[pl.ds(0, 1)],
                              vms[0].at[pl.ds(0, 1)], souts[0]).wait()


def _body(xt_hbm, tbl_hbm, wp_hbm, out_hbm,
          users_v, lb_v, wp_v, idx_v, val_v, o0_v, o1_v, sems):
    wid = lax.axis_index("s") * 2 + lax.axis_index("c")
    base = wid * _NPW

    pltpu.sync_copy(xt_hbm.at[pl.ds(base, _NPW)], users_v.at[pl.ds(0, _NPW)])
    pltpu.sync_copy(xt_hbm.at[pl.ds(_BATCH + base, _NPW)],
                    users_v.at[pl.ds(_NPW, _NPW)])
    pltpu.sync_copy(xt_hbm.at[pl.ds(4 * _BATCH + base, _NPW)], lb_v)
    pltpu.sync_copy(wp_hbm, wp_v)

    def build(g, carry):
        for t in range(8):
            u = users_v[pl.ds(g * _CSZ + t * 16, 16)].astype(jnp.int32)
            v = ((u >> 7) << 10) | (u & 127)
            idx_v[pl.ds(g * _CSZ + t * 16, 16)] = v
        return carry

    lax.fori_loop(0, 2 * _NBLK, build, 0)

    for blk in range(_NBLK):
        for sel in range(2):
            isl = idx_v.at[pl.ds((sel * _NBLK + blk) * _CSZ, _CSZ)]
            for c in range(16):
                cbase = (c >> 3) * _HSTRIDE + (c & 7) * 128
                pltpu.async_copy(
                    tbl_hbm.at[pl.ds(cbase, _WIN)].at[isl],
                    val_v.at[pl.ds(((sel * 16 + c) * _NBLK + blk) * _CSZ,
                                   _CSZ)],
                    sems[blk])

    wlbl = wp_v[pl.ds(32 * 16, 16)]
    wdb = wp_v[pl.ds(33 * 16, 16)]
    wv = [wp_v[pl.ds(c * 16, 16)] for c in range(32)]

    for blk in range(_NBLK):
        pltpu.make_async_copy(xt_hbm.at[pl.ds(0, 32 * _CSZ)],
                              val_v.at[pl.ds(0, 32 * _CSZ)], sems[blk]).wait()
        for t in range(8):
            e = blk * 128 + t * 16
            acc = lb_v[pl.ds(e, 16)] * wlbl + wdb
            for c in range(16):
                vl = val_v[pl.ds((c * _NBLK + blk) * _CSZ + t * 16, 16)]
                acc = acc + vl * wv[c]
            for c in range(16):
                vc = val_v[pl.ds(((16 + c) * _NBLK + blk) * _CSZ + t * 16, 16)]
                acc = acc + vc * wv[16 + c]
            e0 = jnp.exp(jnp.minimum(-acc, 0.0))
            e1 = jnp.exp(jnp.minimum(acc, 0.0))
            rz = 1.0 / (e0 + e1)
            o0_v[pl.ds(e, 16)] = e0 * rz
            o1_v[pl.ds(e, 16)] = e1 * rz

    pltpu.sync_copy(o0_v, out_hbm.at[pl.ds(base, _NPW)])
    pltpu.sync_copy(o1_v, out_hbm.at[pl.ds(_BATCH + base, _NPW)])


@functools.partial(jax.jit, static_argnums=())
def _run(xt_flat, tbl_t, wp):
    mesh = plsc.VectorSubcoreMesh(core_axis_name="c", subcore_axis_name="s")
    retile = pl.kernel(
        _retile,
        out_type=jax.ShapeDtypeStruct((_NTILES, 8, 128), jnp.float32),
        mesh=mesh,
        scratch_types=[
            [pltpu.VMEM((_CK, 8, 128), jnp.float32) for _ in range(_NBUF)],
            [pltpu.SemaphoreType.DMA for _ in range(_NBUF)],
            [pltpu.SemaphoreType.DMA for _ in range(_NBUF)],
        ],
        compiler_params=pltpu.CompilerParams(use_tc_tiling_on_sc=True),
    )
    t2 = retile(tbl_t)
    f = pl.kernel(
        _body,
        out_type=jax.ShapeDtypeStruct((2 * _BATCH,), jnp.float32),
        mesh=mesh,
        scratch_types=[
            pltpu.VMEM((2 * _NPW,), jnp.float32),
            pltpu.VMEM((_NPW,), jnp.float32),
            pltpu.VMEM((34 * 16,), jnp.float32),
            pltpu.VMEM((2 * _NPW,), jnp.int32),
            pltpu.VMEM((32 * _NPW,), jnp.float32),
            pltpu.VMEM((_NPW,), jnp.float32),
            pltpu.VMEM((_NPW,), jnp.float32),
            [pltpu.SemaphoreType.DMA for _ in range(_NBLK)],
        ],
        compiler_params=pltpu.CompilerParams(
            needs_layout_passes=False, use_tc_tiling_on_sc=False),
    )
    return f(xt_flat, t2.reshape(-1), wp)


def kernel(x, table, W, b):
    wd = W[1] - W[0]
    wp = jnp.concatenate([
        jnp.broadcast_to(wd[:32, None], (32, 16)),
        jnp.full((1, 16), wd[32], jnp.float32),
        jnp.full((1, 16), b[1] - b[0], jnp.float32),
    ], axis=0).reshape(-1)
    xt = x.transpose(2, 1, 0).reshape(-1)
    out = _run(xt, table.T, wp)
    return out.reshape(2, _BATCH).transpose(1, 0)
